# Initial kernel scaffold; baseline (speedup 1.0000x reference)
import jax
import jax.numpy as jnp
from jax.experimental import pallas as pl


def kernel(x, logits_table):
    raise NotImplementedError("write your pallas kernel here")


# speedup vs baseline: 3.0466x; 3.0466x over previous
import functools

import jax
import jax.numpy as jnp
from jax import lax
from jax.experimental import pallas as pl
from jax.experimental.pallas import tpu as pltpu
from jax.experimental.pallas import tpu_sc as plsc

EMBED_DIM = 128
CHUNK = 128


@functools.cache
def _build(B: int, V: int, D: int):
    info = plsc.get_sparse_core_info()
    NC, NS = info.num_cores, info.num_subcores
    NW = NC * NS
    assert B % (NW * CHUNK) == 0
    b_per_w = B // NW
    n_chunks = b_per_w // CHUNK
    mesh = plsc.VectorSubcoreMesh(core_axis_name="c", subcore_axis_name="s")

    @functools.partial(
        pl.kernel,
        out_type=jax.ShapeDtypeStruct((B, D), jnp.float32),
        mesh=mesh,
        scratch_types=[
            pltpu.VMEM((n_chunks, CHUNK), jnp.int32),
            pltpu.VMEM((2, CHUNK, D), jnp.float32),
            pltpu.SemaphoreType.DMA,
        ],
    )
    def gather_kernel(idx_hbm, table_hbm, out_hbm, idx_v, rows_v, gsem):
        wid = lax.axis_index("s") * NC + lax.axis_index("c")
        base = wid * b_per_w
        pltpu.sync_copy(idx_hbm.at[wid], idx_v)

        def body(j, carry):
            pltpu.async_copy(
                table_hbm.at[idx_v.at[j]], rows_v.at[0], gsem
            ).wait()
            pltpu.sync_copy(
                rows_v.at[0], out_hbm.at[pl.ds(base + j * CHUNK, CHUNK)]
            )
            return carry

        lax.fori_loop(0, n_chunks, body, 0)

    return gather_kernel, NW, n_chunks


def kernel(x, logits_table):
    BATCH, HIST = x.shape[0], x.shape[1]
    B = BATCH * HIST
    V, D = logits_table.shape
    fn, NW, n_chunks = _build(B, V, D)
    ids = x.reshape(NW, n_chunks, CHUNK).astype(jnp.int32)
    out = fn(ids, logits_table)
    return out.reshape(BATCH, HIST, D)


# speedup vs baseline: 3.4902x; 1.1456x over previous
import functools

import jax
import jax.numpy as jnp
from jax import lax
from jax.experimental import pallas as pl
from jax.experimental.pallas import tpu as pltpu
from jax.experimental.pallas import tpu_sc as plsc

EMBED_DIM = 128
CHUNK = 128
NBUF = 6
OUT_AHEAD = 2


@functools.cache
def _build(B: int, V: int, D: int):
    info = plsc.get_sparse_core_info()
    NC, NS = info.num_cores, info.num_subcores
    NW = NC * NS
    assert B % (NW * CHUNK) == 0
    b_per_w = B // NW
    n_chunks = b_per_w // CHUNK
    assert n_chunks > NBUF
    mesh = plsc.VectorSubcoreMesh(core_axis_name="c", subcore_axis_name="s")

    @functools.partial(
        pl.kernel,
        out_type=jax.ShapeDtypeStruct((B, D), jnp.float32),
        mesh=mesh,
        scratch_types=[
            pltpu.VMEM((n_chunks, CHUNK), jnp.int32),
            pltpu.VMEM((NBUF, CHUNK, D), jnp.float32),
            pltpu.SemaphoreType.DMA,
            pltpu.SemaphoreType.DMA,
        ],
    )
    def gather_kernel(idx_hbm, table_hbm, out_hbm, idx_v, rows_v, gsem, osem):
        wid = lax.axis_index("s") * NC + lax.axis_index("c")
        base = wid * b_per_w
        pltpu.sync_copy(idx_hbm.at[wid], idx_v)

        def start_gather(c):
            pltpu.async_copy(
                table_hbm.at[idx_v.at[c]], rows_v.at[lax.rem(c, NBUF)], gsem
            )

        def wait_gather(slot):
            pltpu.make_async_copy(
                table_hbm.at[pl.ds(0, CHUNK)], rows_v.at[slot], gsem
            ).wait()

        def wait_out():
            pltpu.make_async_copy(
                rows_v.at[0], out_hbm.at[pl.ds(base, CHUNK)], osem
            ).wait()

        for c in range(NBUF):
            start_gather(c)

        def body(j, carry):
            slot = lax.rem(j, NBUF)

            @pl.when(j >= OUT_AHEAD)
            def _():
                wait_out()

            @pl.when(jnp.logical_and(j >= OUT_AHEAD, j + NBUF - OUT_AHEAD < n_chunks))
            def _():
                start_gather(j + NBUF - OUT_AHEAD)

            wait_gather(slot)
            pltpu.async_copy(
                rows_v.at[slot], out_hbm.at[pl.ds(base + j * CHUNK, CHUNK)], osem
            )
            return carry

        lax.fori_loop(0, n_chunks, body, 0)
        for _ in range(OUT_AHEAD):
            wait_out()

    return gather_kernel, NW, n_chunks


def kernel(x, logits_table):
    BATCH, HIST = x.shape[0], x.shape[1]
    B = BATCH * HIST
    V, D = logits_table.shape
    fn, NW, n_chunks = _build(B, V, D)
    ids = x.reshape(NW, n_chunks, CHUNK).astype(jnp.int32)
    out = fn(ids, logits_table)
    return out.reshape(BATCH, HIST, D)


# speedup vs baseline: 5.9172x; 1.6954x over previous
import functools

import jax
import jax.numpy as jnp
from jax import lax
from jax.experimental import pallas as pl
from jax.experimental.pallas import tpu as pltpu
from jax.experimental.pallas import tpu_sc as plsc

ENT = 4
NBUF = 8
OUT_AHEAD = 2


@functools.cache
def _build(BATCH: int, HIST: int, V: int, D: int):
    info = plsc.get_sparse_core_info()
    NC, NS = info.num_cores, info.num_subcores
    NW = NC * NS
    CH_ROWS = ENT * HIST
    assert CH_ROWS <= 128 and BATCH % (NW * ENT) == 0
    e_per_w = BATCH // NW
    b_per_w = e_per_w * HIST
    n_chunks = e_per_w // ENT
    assert n_chunks > NBUF
    mesh = plsc.VectorSubcoreMesh(core_axis_name="c", subcore_axis_name="s")

    @functools.partial(
        pl.kernel,
        out_type=jax.ShapeDtypeStruct((BATCH, HIST, D), jnp.float32),
        mesh=mesh,
        scratch_types=[
            pltpu.VMEM((b_per_w,), jnp.int32),
            pltpu.VMEM((NBUF, CH_ROWS, D), jnp.float32),
            pltpu.SemaphoreType.DMA,
            pltpu.SemaphoreType.DMA,
        ],
        compiler_params=pltpu.CompilerParams(use_tc_tiling_on_sc=True),
    )
    def gather_kernel(idx_hbm, table_hbm, out_hbm, idx_v, rows_v, gsem, osem):
        wid = lax.axis_index("s") * NC + lax.axis_index("c")
        ebase = wid * e_per_w
        pltpu.sync_copy(idx_hbm.at[wid], idx_v)

        def start_gather(c):
            pltpu.async_copy(
                table_hbm.at[idx_v.at[pl.ds(c * CH_ROWS, CH_ROWS)]],
                rows_v.at[lax.rem(c, NBUF)],
                gsem,
            )

        def wait_gather(slot):
            pltpu.make_async_copy(
                table_hbm.at[pl.ds(0, CH_ROWS)], rows_v.at[slot], gsem
            ).wait()

        def wait_outs():
            for _ in range(ENT):
                pltpu.make_async_copy(
                    rows_v.at[0, pl.ds(0, HIST)], out_hbm.at[0], osem
                ).wait()

        for c in range(NBUF):
            start_gather(c)

        def body(j, carry):
            slot = lax.rem(j, NBUF)

            @pl.when(j >= OUT_AHEAD)
            def _():
                wait_outs()

            @pl.when(jnp.logical_and(j >= OUT_AHEAD, j + NBUF - OUT_AHEAD < n_chunks))
            def _():
                start_gather(j + NBUF - OUT_AHEAD)

            wait_gather(slot)
            for e in range(ENT):
                pltpu.async_copy(
                    rows_v.at[slot, pl.ds(e * HIST, HIST)],
                    out_hbm.at[ebase + j * ENT + e],
                    osem,
                )
            return carry

        lax.fori_loop(0, n_chunks, body, 0)
        for _ in range(OUT_AHEAD):
            wait_outs()

    return gather_kernel, NW


def kernel(x, logits_table):
    BATCH, HIST = x.shape[0], x.shape[1]
    V, D = logits_table.shape
    fn, NW = _build(BATCH, HIST, V, D)
    ids = x.reshape(NW, (BATCH // NW) * HIST).astype(jnp.int32)
    return fn(ids, logits_table)


# speedup vs baseline: 11.4852x; 1.9410x over previous
import functools

import jax
import jax.numpy as jnp
from jax import lax
from jax.experimental import pallas as pl
from jax.experimental.pallas import tpu as pltpu
from jax.experimental.pallas import tpu_sc as plsc

CHUNK = 128
NBUF = 6
OUT_AHEAD = 2


@functools.cache
def _build(B: int, V: int, D: int):
    info = plsc.get_sparse_core_info()
    NC, NS = info.num_cores, info.num_subcores
    NW = NC * NS
    assert B % (NW * CHUNK) == 0
    b_per_w = B // NW
    n_chunks = b_per_w // CHUNK
    assert n_chunks > NBUF
    mesh = plsc.VectorSubcoreMesh(core_axis_name="c", subcore_axis_name="s")

    @functools.partial(
        pl.kernel,
        out_type=jax.ShapeDtypeStruct((B, D), jnp.float32),
        mesh=mesh,
        scratch_types=[
            pltpu.VMEM((n_chunks, CHUNK), jnp.int32),
            pltpu.VMEM((NBUF, CHUNK, D), jnp.float32),
            pltpu.SemaphoreType.DMA,
            pltpu.SemaphoreType.DMA,
        ],
    )
    def gather_kernel(idx_hbm, table_hbm, out_hbm, idx_v, rows_v, gsem, osem):
        wid = lax.axis_index("s") * NC + lax.axis_index("c")
        base = wid * b_per_w
        pltpu.sync_copy(idx_hbm.at[wid], idx_v)

        def start_gather(c):
            pltpu.async_copy(
                table_hbm.at[idx_v.at[c]], rows_v.at[lax.rem(c, NBUF)], gsem
            )

        def wait_gather(slot):
            pltpu.make_async_copy(
                table_hbm.at[pl.ds(0, CHUNK)], rows_v.at[slot], gsem
            ).wait()

        def wait_out():
            pltpu.make_async_copy(
                rows_v.at[0], out_hbm.at[pl.ds(base, CHUNK)], osem
            ).wait()

        for c in range(NBUF):
            start_gather(c)

        def body(j, carry):
            slot = lax.rem(j, NBUF)

            @pl.when(j >= OUT_AHEAD)
            def _():
                wait_out()

            @pl.when(jnp.logical_and(j >= OUT_AHEAD, j + NBUF - OUT_AHEAD < n_chunks))
            def _():
                start_gather(j + NBUF - OUT_AHEAD)

            wait_gather(slot)
            pltpu.async_copy(
                rows_v.at[slot], out_hbm.at[pl.ds(base + j * CHUNK, CHUNK)], osem
            )
            return carry

        lax.fori_loop(0, n_chunks, body, 0)
        for _ in range(OUT_AHEAD):
            wait_out()

    return gather_kernel, NW, n_chunks


def kernel(x, logits_table):
    BATCH, HIST = x.shape[0], x.shape[1]
    B = BATCH * HIST
    V, D = logits_table.shape
    fn, NW, n_chunks = _build(B, V, D)
    ids = jnp.swapaxes(x.reshape(BATCH, HIST), 0, 1)
    ids = ids.reshape(NW, n_chunks, CHUNK).astype(jnp.int32)
    out = fn(ids, logits_table)
    return jnp.swapaxes(out.reshape(HIST, BATCH, D), 0, 1)
